# Initial kernel scaffold; baseline (speedup 1.0000x reference)
import jax
import jax.numpy as jnp
from jax.experimental import pallas as pl


def kernel(x, edge_index, batch, params):
    raise NotImplementedError("write your pallas kernel here")


# speedup vs baseline: 5.5016x; 5.5016x over previous
import functools

import jax
import jax.numpy as jnp
from jax import lax
from jax.experimental import pallas as pl
from jax.experimental.pallas import tpu as pltpu
from jax.experimental.pallas import tpu_sc as plsc

N_NODES = 10000
N_EDGES = 320000
DIM = 128
N_GRAPHS = 16
OUT_CH = 10

NC = 2
NS = 16
NW = NC * NS
EPW = N_EDGES // NW
CHUNK = 80
NCHUNK = EPW // CHUNK
RPT = 624
TAIL = N_NODES - NS * RPT
ZROWS = 208

_LANES = 16


def _agg_body(h_hbm, src_hbm, dst_hbm, out_hbm,
              acc, src_v, dst_v, rows_v, zbuf, sem):
    c = lax.axis_index("c")
    s = lax.axis_index("s")
    wid = c * NS + s

    @pl.loop(0, ZROWS)
    def _zero(r):
        for j in range(0, DIM, _LANES):
            zbuf[r, pl.ds(j, _LANES)] = jnp.zeros((_LANES,), jnp.float32)

    @pl.loop(0, RPT // ZROWS)
    def _init(j):
        pltpu.sync_copy(zbuf, acc.at[pl.ds(s * RPT + j * ZROWS, ZROWS)])

    @pl.when(s == NS - 1)
    def _init_tail():
        pltpu.sync_copy(zbuf.at[pl.ds(0, TAIL)],
                        acc.at[pl.ds(NS * RPT, TAIL)])

    pltpu.sync_copy(src_hbm.at[pl.ds(wid * EPW, EPW)], src_v)

    plsc.subcore_barrier()

    @pl.loop(0, NCHUNK)
    def _chunk(t):
        pltpu.sync_copy(dst_hbm.at[pl.ds(wid * EPW + t * CHUNK, CHUNK)], dst_v)
        pltpu.async_copy(
            h_hbm.at[src_v.at[pl.ds(t * CHUNK, CHUNK)]], rows_v, sem).wait()
        pltpu.sync_copy(rows_v, acc.at[dst_v], add=True)

    plsc.subcore_barrier()

    pltpu.sync_copy(acc.at[pl.ds(s * RPT, RPT)],
                    out_hbm.at[c].at[pl.ds(s * RPT, RPT)])

    @pl.when(s == NS - 1)
    def _out_tail():
        pltpu.sync_copy(acc.at[pl.ds(NS * RPT, TAIL)],
                        out_hbm.at[c].at[pl.ds(NS * RPT, TAIL)])


_agg = pl.kernel(
    _agg_body,
    out_type=jax.ShapeDtypeStruct((NC, N_NODES, DIM), jnp.float32),
    mesh=plsc.VectorSubcoreMesh(core_axis_name="c", subcore_axis_name="s"),
    scratch_types=[
        pltpu.VMEM_SHARED((N_NODES, DIM), jnp.float32),
        pltpu.VMEM((EPW,), jnp.int32),
        pltpu.VMEM((CHUNK,), jnp.int32),
        pltpu.VMEM((CHUNK, DIM), jnp.float32),
        pltpu.VMEM((ZROWS, DIM), jnp.float32),
        pltpu.SemaphoreType.DMA,
    ],
)


_HI = lax.Precision.HIGHEST
_RB = 1000
_NRB = N_NODES // _RB


def _layer_math(x_blk, a_ref, w1s_ref, b1s_ref, w2_ref, b2_ref):
    h = x_blk + a_ref[0] + a_ref[1]
    t = jnp.dot(h, w1s_ref[...], precision=_HI) + b1s_ref[...]
    t = jnp.maximum(t, 0.0)
    o = jnp.dot(t, w2_ref[...], precision=_HI) + b2_ref[...]
    return jnp.maximum(o, 0.0)


def _mlp_body(x_ref, a_ref, w1s_ref, b1s_ref, w2_ref, b2_ref, o_ref):
    o_ref[...] = _layer_math(x_ref[...], a_ref, w1s_ref, b1s_ref,
                             w2_ref, b2_ref)


def _mlp(h, agg, w1s, b1s, w2, b2):
    return pl.pallas_call(
        _mlp_body,
        grid=(_NRB,),
        in_specs=[
            pl.BlockSpec((_RB, DIM), lambda i: (i, 0)),
            pl.BlockSpec((NC, _RB, DIM), lambda i: (0, i, 0)),
            pl.BlockSpec((DIM, DIM), lambda i: (0, 0)),
            pl.BlockSpec((1, DIM), lambda i: (0, 0)),
            pl.BlockSpec((DIM, DIM), lambda i: (0, 0)),
            pl.BlockSpec((1, DIM), lambda i: (0, 0)),
        ],
        out_specs=pl.BlockSpec((_RB, DIM), lambda i: (i, 0)),
        out_shape=jax.ShapeDtypeStruct((N_NODES, DIM), jnp.float32),
    )(h, agg, w1s, b1s, w2, b2)


def _head_body(x_ref, a_ref, batch_ref, w1s_ref, b1s_ref, w2_ref, b2_ref,
               l1w_ref, l1b_ref, l2w_ref, l2b_ref, o_ref, pool_acc):
    i = pl.program_id(0)
    h5 = _layer_math(x_ref[...], a_ref, w1s_ref, b1s_ref, w2_ref, b2_ref)
    b = batch_ref[0, 0, :]
    onehot = (b[:, None] == lax.broadcasted_iota(
        jnp.int32, (1, N_GRAPHS), 1)).astype(jnp.float32)
    part = lax.dot_general(onehot, h5, (((0,), (0,)), ((), ())),
                           precision=_HI)

    @pl.when(i == 0)
    def _first():
        pool_acc[...] = part

    @pl.when(i > 0)
    def _rest():
        pool_acc[...] += part

    @pl.when(i == _NRB - 1)
    def _final():
        pooled = pool_acc[...]
        u = jnp.dot(pooled, l1w_ref[...], precision=_HI) + l1b_ref[...]
        u = jnp.maximum(u, 0.0)
        o = jnp.dot(u, l2w_ref[...], precision=_HI) + l2b_ref[...]
        m = jnp.max(o, axis=-1, keepdims=True)
        e = o - m
        o_ref[...] = e - jnp.log(jnp.sum(jnp.exp(e), axis=-1, keepdims=True))


def _head(h, agg, batch_r, w1s, b1s, w2, b2, l1w, l1b, l2w, l2b):
    return pl.pallas_call(
        _head_body,
        grid=(_NRB,),
        in_specs=[
            pl.BlockSpec((_RB, DIM), lambda i: (i, 0)),
            pl.BlockSpec((NC, _RB, DIM), lambda i: (0, i, 0)),
            pl.BlockSpec((1, 1, _RB), lambda i: (i, 0, 0)),
            pl.BlockSpec((DIM, DIM), lambda i: (0, 0)),
            pl.BlockSpec((1, DIM), lambda i: (0, 0)),
            pl.BlockSpec((DIM, DIM), lambda i: (0, 0)),
            pl.BlockSpec((1, DIM), lambda i: (0, 0)),
            pl.BlockSpec((DIM, DIM), lambda i: (0, 0)),
            pl.BlockSpec((1, DIM), lambda i: (0, 0)),
            pl.BlockSpec((DIM, OUT_CH), lambda i: (0, 0)),
            pl.BlockSpec((1, OUT_CH), lambda i: (0, 0)),
        ],
        out_specs=pl.BlockSpec((N_GRAPHS, OUT_CH), lambda i: (0, 0)),
        out_shape=jax.ShapeDtypeStruct((N_GRAPHS, OUT_CH), jnp.float32),
        scratch_shapes=[pltpu.VMEM((N_GRAPHS, DIM), jnp.float32)],
    )(h, agg, batch_r, w1s, b1s, w2, b2, l1w, l1b, l2w, l2b)


def kernel(x, edge_index, batch, params):
    src = edge_index[0].astype(jnp.int32)
    dst = edge_index[1].astype(jnp.int32)
    batch_r = batch.astype(jnp.int32).reshape(_NRB, 1, _RB)

    bn_rsqrt = 1.0 / jnp.sqrt(jnp.float32(1.0 + 1e-5))
    h = x
    out = None
    for li in range(1, 6):
        p = params[f"conv{li}"]
        scale = p["g"] * bn_rsqrt
        w1s = p["W1"] * scale[None, :]
        b1s = (p["b1"] * scale + p["b"]).reshape(1, DIM)
        w2 = p["W2"]
        b2 = p["b2"].reshape(1, DIM)
        agg = _agg(h, src, dst)
        if li < 5:
            h = _mlp(h, agg, w1s, b1s, w2, b2)
        else:
            out = _head(h, agg, batch_r, w1s, b1s, w2, b2,
                        params["lin1_W"], params["lin1_b"].reshape(1, DIM),
                        params["lin2_W"], params["lin2_b"].reshape(1, OUT_CH))
    return out
